# baseline (speedup 1.0000x reference)
import functools

import jax
import jax.numpy as jnp
from jax import lax
from jax.experimental import pallas as pl
from jax.experimental.pallas import tpu as pltpu
from jax.experimental.pallas import tpu_sc as plsc

_B, _C, _N = 2, 16, 8192
_DS = 4
_M = _N // _DS
_K = 32
_XYZN = 7
_EPS = 1e-5
_QT = 128
_ST = 8192
_MT = 256
_TOTAL = _B * _K * _M
_NW = 32


def _knn_kern(pts_ref, q_ref, out_ref, d_ref):
    px = pts_ref[0, 0:1, :]
    py = pts_ref[0, 1:2, :]
    pz = pts_ref[0, 2:3, :]
    qx = q_ref[0, :, 0:1]
    qy = q_ref[0, :, 1:2]
    qz = q_ref[0, :, 2:3]
    d_ref[...] = (qx - px) ** 2 + (qy - py) ** 2 + (qz - pz) ** 2
    iota = lax.broadcasted_iota(jnp.int32, (_QT, _N), 1)

    def body(k, _):
        d = d_ref[...]
        mn = jnp.min(d, axis=1, keepdims=True)
        am = jnp.min(jnp.where(d == mn, iota, _N), axis=1)
        out_ref[0, pl.ds(k, 1), :] = am[None, :]
        d_ref[...] = jnp.where(iota == am[:, None], jnp.inf, d)
        return 0

    lax.fori_loop(0, _K, body, 0)


def _knn(pts, qT):
    return pl.pallas_call(
        _knn_kern,
        grid=(_B, _M // _QT),
        in_specs=[
            pl.BlockSpec((1, 3, _N), lambda b, t: (b, 0, 0)),
            pl.BlockSpec((1, _QT, 3), lambda b, t: (b, t, 0)),
        ],
        out_specs=pl.BlockSpec((1, _K, _QT), lambda b, t: (b, 0, t)),
        out_shape=jax.ShapeDtypeStruct((_B, _K, _M), jnp.int32),
        scratch_shapes=[pltpu.VMEM((_QT, _N), jnp.float32)],
    )(pts, qT)


_QPW = _M * _B // _NW
_QI = 8
_NG = 512
_GV = _NG // 16


def _knn_sc(pts, qprep):
    info = plsc.get_sparse_core_info()
    nc = info.num_cores
    mesh = plsc.VectorSubcoreMesh(core_axis_name="c", subcore_axis_name="s")

    @functools.partial(
        pl.kernel,
        mesh=mesh,
        compiler_params=pltpu.CompilerParams(
            use_tc_tiling_on_sc=False, needs_layout_passes=False),
        out_type=jax.ShapeDtypeStruct((_B * _M * _K,), jnp.int32),
        scratch_types=[
            pltpu.VMEM((3, _N), jnp.float32),
            pltpu.VMEM((3, _QPW), jnp.float32),
            pltpu.VMEM((_QI, _N), jnp.float32),
            pltpu.VMEM((_QI, _NG), jnp.float32),
            pltpu.VMEM((_QI, 32), jnp.float32),
            pltpu.VMEM((_QPW * _K,), jnp.int32),
        ],
    )
    def kk(pts_hbm, q_hbm, out_hbm, ptsv, qv, dbuf, gm, gmm, idxout):
        w = lax.axis_index("s") * nc + lax.axis_index("c")
        b = w // (_NW // _B)
        pltpu.sync_copy(pts_hbm.at[b], ptsv)
        pltpu.sync_copy(q_hbm.at[w], qv)

        iota = lax.broadcasted_iota(jnp.int32, (16,), 0)
        lane0 = iota == 0
        lane1 = iota == 1
        zero16 = jnp.zeros((16,), jnp.int32)
        one16 = jnp.full((16,), 1, jnp.int32)
        two16 = jnp.full((16,), 2, jnp.int32)
        sixteen16 = jnp.full((16,), 16, jnp.int32)
        inf16 = jnp.full((16,), jnp.inf, jnp.float32)
        iota16x = iota * 16
        iota512 = iota * _NG

        def vmin_splat(x):
            nx = -x
            return -plsc.cummax(jnp.flip(plsc.cummax(nx)))

        def per_quad(qp, qis):
            qs = [qis + jnp.full((16,), dq, jnp.int32) for dq in range(_QI)]
            qx = [plsc.load_gather(qv, [zero16, q]) for q in qs]
            qy = [plsc.load_gather(qv, [one16, q]) for q in qs]
            qz = [plsc.load_gather(qv, [two16, q]) for q in qs]

            def dist_chunk(c):
                px = ptsv[0, pl.ds(c * 16, 16)]
                py = ptsv[1, pl.ds(c * 16, 16)]
                pz = ptsv[2, pl.ds(c * 16, 16)]
                ds = []
                for q in range(_QI):
                    dx = qx[q] - px
                    dy = qy[q] - py
                    dz = qz[q] - pz
                    d = dx * dx + dy * dy + dz * dz
                    dbuf[q, pl.ds(c * 16, 16)] = d
                    ds.append(d)
                return tuple(ds)

            def outer(j, vh):
                def inner(k, acc):
                    d = dist_chunk(j + _GV * k)
                    return tuple(jnp.minimum(acc[q], d[q]) for q in range(_QI))

                acc = lax.fori_loop(1, 16, inner, dist_chunk(j), unroll=4)
                for q in range(_QI):
                    gm[q, pl.ds(j * 16, 16)] = acc[q]
                return tuple(jnp.minimum(vh[q], acc[q]) for q in range(_QI))

            va = lax.fori_loop(0, 16, outer, (inf16,) * _QI)
            vb = lax.fori_loop(16, 32, outer, (inf16,) * _QI)
            for q in range(_QI):
                gmm[q, pl.ds(0, 16)] = va[q]
                gmm[q, pl.ds(16, 16)] = vb[q]

            def extract_one(qsel, ks):
                m2a = gmm[qsel, pl.ds(0, 16)]
                m2b = gmm[qsel, pl.ds(16, 16)]
                gmin = vmin_splat(jnp.minimum(m2a, m2b))
                f_a = plsc.all_reduce_ffs(m2a == gmin)
                f_b = plsc.all_reduce_ffs(m2b == gmin)
                isa = f_a < sixteen16
                l2 = jnp.where(isa, f_a, f_b)
                hcell = jnp.where(isa, zero16, sixteen16)
                hbase = hcell * 16
                qsel16 = jnp.full((16,), qsel, jnp.int32)
                gmv = plsc.load_gather(gm, [qsel16, iota16x + hbase + l2])
                jloc = plsc.all_reduce_ffs(gmv == gmin)
                gstar = hbase + jloc * 16 + l2
                midx = iota512 + gstar
                dv = plsc.load_gather(dbuf, [qsel16, midx])
                sd, si = plsc.sort_key_val(dv, midx)
                sgd, _sgi = plsc.sort_key_val(gmv, gmv)
                plsc.store_scatter(idxout, [ks], si, mask=lane0)
                plsc.store_scatter(dbuf, [qsel16, si], inf16, mask=lane0)
                plsc.store_scatter(gm, [qsel16, gstar], sd, mask=lane1)
                plsc.store_scatter(gmm, [qsel16, hcell + l2],
                                   jnp.minimum(sgd, sd), mask=lane1)

            def extract(i, ks):
                for q in range(_QI):
                    extract_one(q, ks + q * _K)
                return ks + 1

            lax.fori_loop(0, _K, extract, qis * _K)
            return qis + _QI

        lax.fori_loop(0, _QPW // _QI, per_quad, zero16)
        pltpu.sync_copy(idxout, out_hbm.at[pl.ds(w * _QPW * _K, _QPW * _K)])

    return kk(pts, qprep)


_CH = 128


def _gather_sc(table, flat_idx):
    per_w = _TOTAL // _NW
    n_ch = per_w // _CH
    info = plsc.get_sparse_core_info()
    nc = info.num_cores
    mesh = plsc.VectorSubcoreMesh(core_axis_name="c", subcore_axis_name="s")

    @functools.partial(
        pl.kernel,
        mesh=mesh,
        compiler_params=pltpu.CompilerParams(
            use_tc_tiling_on_sc=False, needs_layout_passes=False),
        out_type=jax.ShapeDtypeStruct((_TOTAL, _C), jnp.float32),
        scratch_types=[
            pltpu.VMEM((_CH,), jnp.int32),
            pltpu.VMEM((_CH, _C), jnp.float32),
            pltpu.SemaphoreType.DMA,
        ],
    )
    def gk(table_hbm, idx_hbm, out_hbm, idx_v, rows_v, sem):
        wid = lax.axis_index("s") * nc + lax.axis_index("c")

        def body(i, _):
            base = wid * per_w + i * _CH
            pltpu.sync_copy(idx_hbm.at[pl.ds(base, _CH)], idx_v)
            pltpu.async_copy(table_hbm.at[idx_v], rows_v, sem).wait()
            pltpu.sync_copy(rows_v, out_hbm.at[pl.ds(base, _CH)])
            return 0

        lax.fori_loop(0, n_ch, body, 0)

    return gk(table, flat_idx)


_SQ = _ST // _K


def _mlp_kern(v_ref, qT_ref, pd_ref, w1aT_ref, w1xT_ref, g1_ref, b1_ref,
              w2T_ref, g2_ref, b2_ref, o_ref, s1, s2, t1, t2):
    p = pl.program_id(0)
    t = pl.program_id(1)
    h1 = jnp.dot(v_ref[...], w1aT_ref[...], preferred_element_type=jnp.float32)
    pt = jnp.dot(qT_ref[0], w1xT_ref[...], preferred_element_type=jnp.float32)
    h1 = (h1.reshape(_SQ, _K, 64) - pt[:, None, :]).reshape(_ST, 64)

    @pl.when(p == 0)
    def _():
        @pl.when(t == 0)
        def _():
            s1[...] = jnp.zeros_like(s1)
            s2[...] = jnp.zeros_like(s2)

        s1[...] += jnp.sum(h1, axis=0, keepdims=True)
        s2[...] += jnp.sum(h1 * h1, axis=0, keepdims=True)

    @pl.when(p > 0)
    def _():
        mu1 = s1[...] / _TOTAL
        var1 = s2[...] / _TOTAL - mu1 * mu1
        sc1 = g1_ref[...] * lax.rsqrt(var1 + _EPS)
        h1r = jnp.maximum((h1 - mu1) * sc1 + b1_ref[...], 0.0)
        h2 = jnp.dot(h1r, w2T_ref[...], preferred_element_type=jnp.float32)

        @pl.when(p == 1)
        def _():
            @pl.when(t == 0)
            def _():
                t1[...] = jnp.zeros_like(t1)
                t2[...] = jnp.zeros_like(t2)

            t1[...] += jnp.sum(h2, axis=0, keepdims=True)
            t2[...] += jnp.sum(h2 * h2, axis=0, keepdims=True)

        @pl.when(p == 2)
        def _():
            mu2 = t1[...] / _TOTAL
            var2 = t2[...] / _TOTAL - mu2 * mu2
            sc2 = g2_ref[...] * lax.rsqrt(var2 + _EPS)
            h2r = jnp.maximum((h2 - mu2) * sc2 + b2_ref[...], 0.0)
            mx = jnp.max(h2r.reshape(_SQ, _K, 128), axis=1)
            o_ref[0, 0:7, :] = pd_ref[0]
            o_ref[0, 7:135, :] = mx.T


def _mlp(v, qT, pd, w1aT, w1xT, g1r, b1r, w2T, g2r, b2r):
    nt = _TOTAL // _ST
    spb = nt // _B
    return pl.pallas_call(
        _mlp_kern,
        grid=(3, nt),
        in_specs=[
            pl.BlockSpec((_ST, _C), lambda p, s: (s, 0)),
            pl.BlockSpec((1, _SQ, 3), lambda p, s: (s // spb, s % spb, 0)),
            pl.BlockSpec((1, 7, _SQ), lambda p, s: (s // spb, 0, s % spb)),
            pl.BlockSpec((_C, 64), lambda p, s: (0, 0)),
            pl.BlockSpec((3, 64), lambda p, s: (0, 0)),
            pl.BlockSpec((1, 64), lambda p, s: (0, 0)),
            pl.BlockSpec((1, 64), lambda p, s: (0, 0)),
            pl.BlockSpec((64, 128), lambda p, s: (0, 0)),
            pl.BlockSpec((1, 128), lambda p, s: (0, 0)),
            pl.BlockSpec((1, 128), lambda p, s: (0, 0)),
        ],
        out_specs=pl.BlockSpec(
            (1, 135, _SQ),
            lambda p, s: (jnp.where(p < 2, 0, s // spb), 0,
                          jnp.where(p < 2, 0, s % spb))),
        out_shape=jax.ShapeDtypeStruct((_B, 135, _M), jnp.float32),
        scratch_shapes=[
            pltpu.VMEM((1, 64), jnp.float32),
            pltpu.VMEM((1, 64), jnp.float32),
            pltpu.VMEM((1, 128), jnp.float32),
            pltpu.VMEM((1, 128), jnp.float32),
        ],
    )(v, qT, pd, w1aT, w1xT, g1r, b1r, w2T, g2r, b2r)


def kernel(x, W1, g1, b1, W2, g2, b2):
    x3 = x[:, :, :, 0]
    pts = x3[:, 0:3, :]
    qc = x3[:, 0:3, ::_DS]
    qT = jnp.transpose(qc, (0, 2, 1))
    qprep = (qc.reshape(_B, 3, _NW // _B, _QPW)
             .transpose(0, 2, 1, 3).reshape(_NW, 3, _QPW))

    idx = _knn_sc(pts, qprep)

    table = jnp.transpose(x3, (0, 2, 1)).reshape(_B * _N, _C)
    flat_idx = (idx.reshape(_B, _M * _K)
                + (jnp.arange(_B, dtype=jnp.int32) * _N)[:, None]).reshape(-1)
    v = _gather_sc(table, flat_idx)

    w1a = jnp.concatenate(
        [W1[:, 0:6], jnp.zeros((64, 1), jnp.float32), W1[:, 6:15]], axis=1)
    w1aT = w1a.T
    w1xT = W1[:, 0:3].T
    g1r, b1r = g1.reshape(1, 64), b1.reshape(1, 64)
    g2r, b2r = g2.reshape(1, 128), b2.reshape(1, 128)
    w2T = W2.T

    pd = x3[:, 0:_XYZN, ::_DS]
    o = _mlp(v, qT, pd, w1aT, w1xT, g1r, b1r, w2T, g2r, b2r)
    return o[..., None]


# speedup vs baseline: 1.1063x; 1.1063x over previous
import functools

import jax
import jax.numpy as jnp
from jax import lax
from jax.experimental import pallas as pl
from jax.experimental.pallas import tpu as pltpu
from jax.experimental.pallas import tpu_sc as plsc

_B, _C, _N = 2, 16, 8192
_DS = 4
_M = _N // _DS
_K = 32
_XYZN = 7
_EPS = 1e-5
_ST = 8192
_TOTAL = _B * _K * _M
_NW = 32


_QPW = _M * _B // _NW
_QI = 4
_NG = 512
_GV = _NG // 16


def _knn_sc(pts, qprep):
    info = plsc.get_sparse_core_info()
    nc = info.num_cores
    mesh = plsc.VectorSubcoreMesh(core_axis_name="c", subcore_axis_name="s")

    @functools.partial(
        pl.kernel,
        mesh=mesh,
        compiler_params=pltpu.CompilerParams(
            use_tc_tiling_on_sc=False, needs_layout_passes=False),
        out_type=jax.ShapeDtypeStruct((_B * _M * _K,), jnp.int32),
        scratch_types=[
            pltpu.VMEM((3, _N), jnp.float32),
            pltpu.VMEM((3, _QPW), jnp.float32),
            pltpu.VMEM((_QI, _N), jnp.float32),
            pltpu.VMEM((_QI, _NG), jnp.float32),
            pltpu.VMEM((_QI, 32), jnp.float32),
            pltpu.VMEM((_QPW * _K,), jnp.int32),
        ],
    )
    def kk(pts_hbm, q_hbm, out_hbm, ptsv, qv, dbuf, gm, gmm, idxout):
        w = lax.axis_index("s") * nc + lax.axis_index("c")
        b = w // (_NW // _B)
        pltpu.sync_copy(pts_hbm.at[b], ptsv)
        pltpu.sync_copy(q_hbm.at[w], qv)

        iota = lax.broadcasted_iota(jnp.int32, (16,), 0)
        lane0 = iota == 0
        lane1 = iota == 1
        zero16 = jnp.zeros((16,), jnp.int32)
        one16 = jnp.full((16,), 1, jnp.int32)
        two16 = jnp.full((16,), 2, jnp.int32)
        sixteen16 = jnp.full((16,), 16, jnp.int32)
        inf16 = jnp.full((16,), jnp.inf, jnp.float32)
        iota16x = iota * 16
        iota512 = iota * _NG

        def vmin_splat(x):
            nx = -x
            return -plsc.cummax(jnp.flip(plsc.cummax(nx)))

        def per_quad(qp, qis):
            qs = [qis + jnp.full((16,), dq, jnp.int32) for dq in range(_QI)]
            qx = [plsc.load_gather(qv, [zero16, q]) for q in qs]
            qy = [plsc.load_gather(qv, [one16, q]) for q in qs]
            qz = [plsc.load_gather(qv, [two16, q]) for q in qs]

            def dist_chunk(c):
                px = ptsv[0, pl.ds(c * 16, 16)]
                py = ptsv[1, pl.ds(c * 16, 16)]
                pz = ptsv[2, pl.ds(c * 16, 16)]
                ds = []
                for q in range(_QI):
                    dx = qx[q] - px
                    dy = qy[q] - py
                    dz = qz[q] - pz
                    d = dx * dx + dy * dy + dz * dz
                    dbuf[q, pl.ds(c * 16, 16)] = d
                    ds.append(d)
                return tuple(ds)

            def outer(j, vh):
                def inner(k, acc):
                    d = dist_chunk(j + _GV * k)
                    return tuple(jnp.minimum(acc[q], d[q]) for q in range(_QI))

                acc = lax.fori_loop(1, 16, inner, dist_chunk(j), unroll=4)
                for q in range(_QI):
                    gm[q, pl.ds(j * 16, 16)] = acc[q]
                return tuple(jnp.minimum(vh[q], acc[q]) for q in range(_QI))

            va = lax.fori_loop(0, 16, outer, (inf16,) * _QI)
            vb = lax.fori_loop(16, 32, outer, (inf16,) * _QI)
            for q in range(_QI):
                gmm[q, pl.ds(0, 16)] = va[q]
                gmm[q, pl.ds(16, 16)] = vb[q]

            def extract_one(qsel, ks):
                m2a = gmm[qsel, pl.ds(0, 16)]
                m2b = gmm[qsel, pl.ds(16, 16)]
                gmin = vmin_splat(jnp.minimum(m2a, m2b))
                f_a = plsc.all_reduce_ffs(m2a == gmin)
                f_b = plsc.all_reduce_ffs(m2b == gmin)
                isa = f_a < sixteen16
                l2 = jnp.where(isa, f_a, f_b)
                hcell = jnp.where(isa, zero16, sixteen16)
                hbase = hcell * 16
                qsel16 = jnp.full((16,), qsel, jnp.int32)
                gmv = plsc.load_gather(gm, [qsel16, iota16x + hbase + l2])
                jloc = plsc.all_reduce_ffs(gmv == gmin)
                gstar = hbase + jloc * 16 + l2
                midx = iota512 + gstar
                dv = plsc.load_gather(dbuf, [qsel16, midx])
                sd, si = plsc.sort_key_val(dv, midx)
                sgd, _sgi = plsc.sort_key_val(gmv, gmv)
                plsc.store_scatter(idxout, [ks], si, mask=lane0)
                plsc.store_scatter(dbuf, [qsel16, si], inf16, mask=lane0)
                plsc.store_scatter(gm, [qsel16, gstar], sd, mask=lane1)
                plsc.store_scatter(gmm, [qsel16, hcell + l2],
                                   jnp.minimum(sgd, sd), mask=lane1)

            def extract(i, ks):
                for q in range(_QI):
                    extract_one(q, ks + q * _K)
                return ks + 1

            lax.fori_loop(0, _K, extract, qis * _K)
            return qis + _QI

        lax.fori_loop(0, _QPW // _QI, per_quad, zero16)
        pltpu.sync_copy(idxout, out_hbm.at[pl.ds(w * _QPW * _K, _QPW * _K)])

    return kk(pts, qprep)


_CH = 128


def _gather_sc(table, flat_idx):
    per_w = _TOTAL // _NW
    n_ch = per_w // _CH
    info = plsc.get_sparse_core_info()
    nc = info.num_cores
    mesh = plsc.VectorSubcoreMesh(core_axis_name="c", subcore_axis_name="s")

    @functools.partial(
        pl.kernel,
        mesh=mesh,
        compiler_params=pltpu.CompilerParams(
            use_tc_tiling_on_sc=False, needs_layout_passes=False),
        out_type=jax.ShapeDtypeStruct((_TOTAL, _C), jnp.float32),
        scratch_types=[
            pltpu.VMEM((_CH,), jnp.int32),
            pltpu.VMEM((_CH, _C), jnp.float32),
            pltpu.SemaphoreType.DMA,
        ],
    )
    def gk(table_hbm, idx_hbm, out_hbm, idx_v, rows_v, sem):
        wid = lax.axis_index("s") * nc + lax.axis_index("c")

        def body(i, _):
            base = wid * per_w + i * _CH
            pltpu.sync_copy(idx_hbm.at[pl.ds(base, _CH)], idx_v)
            pltpu.async_copy(table_hbm.at[idx_v], rows_v, sem).wait()
            pltpu.sync_copy(rows_v, out_hbm.at[pl.ds(base, _CH)])
            return 0

        lax.fori_loop(0, n_ch, body, 0)

    return gk(table, flat_idx)


_SQ = _ST // _K


def _mlp_kern(v_ref, qT_ref, pd_ref, w1aT_ref, w1xT_ref, g1_ref, b1_ref,
              w2T_ref, g2_ref, b2_ref, o_ref, s1, s2, t1, t2):
    p = pl.program_id(0)
    t = pl.program_id(1)
    h1 = jnp.dot(v_ref[...], w1aT_ref[...], preferred_element_type=jnp.float32)
    pt = jnp.dot(qT_ref[0], w1xT_ref[...], preferred_element_type=jnp.float32)
    h1 = (h1.reshape(_SQ, _K, 64) - pt[:, None, :]).reshape(_ST, 64)

    @pl.when(p == 0)
    def _():
        @pl.when(t == 0)
        def _():
            s1[...] = jnp.zeros_like(s1)
            s2[...] = jnp.zeros_like(s2)

        s1[...] += jnp.sum(h1, axis=0, keepdims=True)
        s2[...] += jnp.sum(h1 * h1, axis=0, keepdims=True)

    @pl.when(p > 0)
    def _():
        mu1 = s1[...] / _TOTAL
        var1 = s2[...] / _TOTAL - mu1 * mu1
        sc1 = g1_ref[...] * lax.rsqrt(var1 + _EPS)
        h1r = jnp.maximum((h1 - mu1) * sc1 + b1_ref[...], 0.0)
        h2 = jnp.dot(h1r, w2T_ref[...], preferred_element_type=jnp.float32)

        @pl.when(p == 1)
        def _():
            @pl.when(t == 0)
            def _():
                t1[...] = jnp.zeros_like(t1)
                t2[...] = jnp.zeros_like(t2)

            t1[...] += jnp.sum(h2, axis=0, keepdims=True)
            t2[...] += jnp.sum(h2 * h2, axis=0, keepdims=True)

        @pl.when(p == 2)
        def _():
            mu2 = t1[...] / _TOTAL
            var2 = t2[...] / _TOTAL - mu2 * mu2
            sc2 = g2_ref[...] * lax.rsqrt(var2 + _EPS)
            h2r = jnp.maximum((h2 - mu2) * sc2 + b2_ref[...], 0.0)
            mx = jnp.max(h2r.reshape(_SQ, _K, 128), axis=1)
            o_ref[0, 0:7, :] = pd_ref[0]
            o_ref[0, 7:135, :] = mx.T


def _mlp(v, qT, pd, w1aT, w1xT, g1r, b1r, w2T, g2r, b2r):
    nt = _TOTAL // _ST
    spb = nt // _B
    return pl.pallas_call(
        _mlp_kern,
        grid=(3, nt),
        in_specs=[
            pl.BlockSpec((_ST, _C), lambda p, s: (s, 0)),
            pl.BlockSpec((1, _SQ, 3), lambda p, s: (s // spb, s % spb, 0)),
            pl.BlockSpec((1, 7, _SQ), lambda p, s: (s // spb, 0, s % spb)),
            pl.BlockSpec((_C, 64), lambda p, s: (0, 0)),
            pl.BlockSpec((3, 64), lambda p, s: (0, 0)),
            pl.BlockSpec((1, 64), lambda p, s: (0, 0)),
            pl.BlockSpec((1, 64), lambda p, s: (0, 0)),
            pl.BlockSpec((64, 128), lambda p, s: (0, 0)),
            pl.BlockSpec((1, 128), lambda p, s: (0, 0)),
            pl.BlockSpec((1, 128), lambda p, s: (0, 0)),
        ],
        out_specs=pl.BlockSpec(
            (1, 135, _SQ),
            lambda p, s: (jnp.where(p < 2, 0, s // spb), 0,
                          jnp.where(p < 2, 0, s % spb))),
        out_shape=jax.ShapeDtypeStruct((_B, 135, _M), jnp.float32),
        scratch_shapes=[
            pltpu.VMEM((1, 64), jnp.float32),
            pltpu.VMEM((1, 64), jnp.float32),
            pltpu.VMEM((1, 128), jnp.float32),
            pltpu.VMEM((1, 128), jnp.float32),
        ],
    )(v, qT, pd, w1aT, w1xT, g1r, b1r, w2T, g2r, b2r)


def kernel(x, W1, g1, b1, W2, g2, b2):
    x3 = x[:, :, :, 0]
    pts = x3[:, 0:3, :]
    qc = x3[:, 0:3, ::_DS]
    qT = jnp.transpose(qc, (0, 2, 1))
    qprep = (qc.reshape(_B, 3, _NW // _B, _QPW)
             .transpose(0, 2, 1, 3).reshape(_NW, 3, _QPW))

    idx = _knn_sc(pts, qprep)

    table = jnp.transpose(x3, (0, 2, 1)).reshape(_B * _N, _C)
    flat_idx = (idx.reshape(_B, _M * _K)
                + (jnp.arange(_B, dtype=jnp.int32) * _N)[:, None]).reshape(-1)
    v = _gather_sc(table, flat_idx)

    w1a = jnp.concatenate(
        [W1[:, 0:6], jnp.zeros((64, 1), jnp.float32), W1[:, 6:15]], axis=1)
    w1aT = w1a.T
    w1xT = W1[:, 0:3].T
    g1r, b1r = g1.reshape(1, 64), b1.reshape(1, 64)
    g2r, b2r = g2.reshape(1, 128), b2.reshape(1, 128)
    w2T = W2.T

    pd = x3[:, 0:_XYZN, ::_DS]
    o = _mlp(v, qT, pd, w1aT, w1xT, g1r, b1r, w2T, g2r, b2r)
    return o[..., None]


# speedup vs baseline: 1.1194x; 1.0119x over previous
import functools

import jax
import jax.numpy as jnp
from jax import lax
from jax.experimental import pallas as pl
from jax.experimental.pallas import tpu as pltpu
from jax.experimental.pallas import tpu_sc as plsc

_B, _C, _N = 2, 16, 8192
_DS = 4
_M = _N // _DS
_K = 32
_XYZN = 7
_EPS = 1e-5
_ST = 16384
_TOTAL = _B * _K * _M
_NW = 32


_QPW = _M * _B // _NW
_QI = 4
_NG = 512
_GV = _NG // 16


def _knn_sc(pts, qprep):
    info = plsc.get_sparse_core_info()
    nc = info.num_cores
    mesh = plsc.VectorSubcoreMesh(core_axis_name="c", subcore_axis_name="s")

    @functools.partial(
        pl.kernel,
        mesh=mesh,
        compiler_params=pltpu.CompilerParams(
            use_tc_tiling_on_sc=False, needs_layout_passes=False),
        out_type=jax.ShapeDtypeStruct((_B * _M * _K,), jnp.int32),
        scratch_types=[
            pltpu.VMEM((3, _N), jnp.float32),
            pltpu.VMEM((3, _QPW), jnp.float32),
            pltpu.VMEM((_QI, _N), jnp.float32),
            pltpu.VMEM((_QI, _NG), jnp.float32),
            pltpu.VMEM((_QI, 32), jnp.float32),
            pltpu.VMEM((_QPW * _K,), jnp.int32),
        ],
    )
    def kk(pts_hbm, q_hbm, out_hbm, ptsv, qv, dbuf, gm, gmm, idxout):
        w = lax.axis_index("s") * nc + lax.axis_index("c")
        b = w // (_NW // _B)
        pltpu.sync_copy(pts_hbm.at[b], ptsv)
        pltpu.sync_copy(q_hbm.at[w], qv)

        iota = lax.broadcasted_iota(jnp.int32, (16,), 0)
        lane0 = iota == 0
        lane1 = iota == 1
        zero16 = jnp.zeros((16,), jnp.int32)
        one16 = jnp.full((16,), 1, jnp.int32)
        two16 = jnp.full((16,), 2, jnp.int32)
        sixteen16 = jnp.full((16,), 16, jnp.int32)
        inf16 = jnp.full((16,), jnp.inf, jnp.float32)
        iota16x = iota * 16
        iota512 = iota * _NG

        def vmin_splat(x):
            nx = -x
            return -plsc.cummax(jnp.flip(plsc.cummax(nx)))

        def per_quad(qp, qis):
            qs = [qis + jnp.full((16,), dq, jnp.int32) for dq in range(_QI)]
            qx = [plsc.load_gather(qv, [zero16, q]) for q in qs]
            qy = [plsc.load_gather(qv, [one16, q]) for q in qs]
            qz = [plsc.load_gather(qv, [two16, q]) for q in qs]

            def dist_chunk(c):
                px = ptsv[0, pl.ds(c * 16, 16)]
                py = ptsv[1, pl.ds(c * 16, 16)]
                pz = ptsv[2, pl.ds(c * 16, 16)]
                ds = []
                for q in range(_QI):
                    dx = qx[q] - px
                    dy = qy[q] - py
                    dz = qz[q] - pz
                    d = dx * dx + dy * dy + dz * dz
                    dbuf[q, pl.ds(c * 16, 16)] = d
                    ds.append(d)
                return tuple(ds)

            def outer(j, vh):
                def inner(k, acc):
                    d = dist_chunk(j + _GV * k)
                    return tuple(jnp.minimum(acc[q], d[q]) for q in range(_QI))

                acc = lax.fori_loop(1, 16, inner, dist_chunk(j), unroll=4)
                for q in range(_QI):
                    gm[q, pl.ds(j * 16, 16)] = acc[q]
                return tuple(jnp.minimum(vh[q], acc[q]) for q in range(_QI))

            va = lax.fori_loop(0, 16, outer, (inf16,) * _QI)
            vb = lax.fori_loop(16, 32, outer, (inf16,) * _QI)
            for q in range(_QI):
                gmm[q, pl.ds(0, 16)] = va[q]
                gmm[q, pl.ds(16, 16)] = vb[q]

            def extract_one(qsel, ks):
                m2a = gmm[qsel, pl.ds(0, 16)]
                m2b = gmm[qsel, pl.ds(16, 16)]
                gmin = vmin_splat(jnp.minimum(m2a, m2b))
                f_a = plsc.all_reduce_ffs(m2a == gmin)
                f_b = plsc.all_reduce_ffs(m2b == gmin)
                isa = f_a < sixteen16
                l2 = jnp.where(isa, f_a, f_b)
                hcell = jnp.where(isa, zero16, sixteen16)
                hbase = hcell * 16
                qsel16 = jnp.full((16,), qsel, jnp.int32)
                gmv = plsc.load_gather(gm, [qsel16, iota16x + hbase + l2])
                jloc = plsc.all_reduce_ffs(gmv == gmin)
                gstar = hbase + jloc * 16 + l2
                midx = iota512 + gstar
                dv = plsc.load_gather(dbuf, [qsel16, midx])
                sd, si = plsc.sort_key_val(dv, midx)
                sgd, _sgi = plsc.sort_key_val(gmv, gmv)
                plsc.store_scatter(idxout, [ks], si, mask=lane0)
                plsc.store_scatter(dbuf, [qsel16, si], inf16, mask=lane0)
                plsc.store_scatter(gm, [qsel16, gstar], sd, mask=lane1)
                plsc.store_scatter(gmm, [qsel16, hcell + l2],
                                   jnp.minimum(sgd, sd), mask=lane1)

            def extract(i, ks):
                for q in range(_QI):
                    extract_one(q, ks + q * _K)
                return ks + 1

            lax.fori_loop(0, _K, extract, qis * _K, unroll=2)
            return qis + _QI

        lax.fori_loop(0, _QPW // _QI, per_quad, zero16)
        pltpu.sync_copy(idxout, out_hbm.at[pl.ds(w * _QPW * _K, _QPW * _K)])

    return kk(pts, qprep)


_CH = 128


def _gather_sc(table, flat_idx):
    per_w = _TOTAL // _NW
    n_ch = per_w // _CH
    info = plsc.get_sparse_core_info()
    nc = info.num_cores
    mesh = plsc.VectorSubcoreMesh(core_axis_name="c", subcore_axis_name="s")

    @functools.partial(
        pl.kernel,
        mesh=mesh,
        compiler_params=pltpu.CompilerParams(
            use_tc_tiling_on_sc=False, needs_layout_passes=False),
        out_type=jax.ShapeDtypeStruct((_TOTAL, _C), jnp.float32),
        scratch_types=[
            pltpu.VMEM((_CH,), jnp.int32),
            pltpu.VMEM((_CH, _C), jnp.float32),
            pltpu.SemaphoreType.DMA,
        ],
    )
    def gk(table_hbm, idx_hbm, out_hbm, idx_v, rows_v, sem):
        wid = lax.axis_index("s") * nc + lax.axis_index("c")

        def body(i, _):
            base = wid * per_w + i * _CH
            pltpu.sync_copy(idx_hbm.at[pl.ds(base, _CH)], idx_v)
            pltpu.async_copy(table_hbm.at[idx_v], rows_v, sem).wait()
            pltpu.sync_copy(rows_v, out_hbm.at[pl.ds(base, _CH)])
            return 0

        lax.fori_loop(0, n_ch, body, 0)

    return gk(table, flat_idx)


_SQ = _ST // _K


def _mlp_kern(v_ref, qT_ref, pd_ref, w1aT_ref, w1xT_ref, g1_ref, b1_ref,
              w2T_ref, g2_ref, b2_ref, o_ref, s1, s2, t1, t2):
    p = pl.program_id(0)
    t = pl.program_id(1)
    h1 = jnp.dot(v_ref[...], w1aT_ref[...], preferred_element_type=jnp.float32)
    pt = jnp.dot(qT_ref[0], w1xT_ref[...], preferred_element_type=jnp.float32)
    h1 = (h1.reshape(_SQ, _K, 64) - pt[:, None, :]).reshape(_ST, 64)

    @pl.when(p == 0)
    def _():
        @pl.when(t == 0)
        def _():
            s1[...] = jnp.zeros_like(s1)
            s2[...] = jnp.zeros_like(s2)

        s1[...] += jnp.sum(h1, axis=0, keepdims=True)
        s2[...] += jnp.sum(h1 * h1, axis=0, keepdims=True)

    @pl.when(p > 0)
    def _():
        mu1 = s1[...] / _TOTAL
        var1 = s2[...] / _TOTAL - mu1 * mu1
        sc1 = g1_ref[...] * lax.rsqrt(var1 + _EPS)
        h1r = jnp.maximum((h1 - mu1) * sc1 + b1_ref[...], 0.0)
        h2 = jnp.dot(h1r, w2T_ref[...], preferred_element_type=jnp.float32)

        @pl.when(p == 1)
        def _():
            @pl.when(t == 0)
            def _():
                t1[...] = jnp.zeros_like(t1)
                t2[...] = jnp.zeros_like(t2)

            t1[...] += jnp.sum(h2, axis=0, keepdims=True)
            t2[...] += jnp.sum(h2 * h2, axis=0, keepdims=True)

        @pl.when(p == 2)
        def _():
            mu2 = t1[...] / _TOTAL
            var2 = t2[...] / _TOTAL - mu2 * mu2
            sc2 = g2_ref[...] * lax.rsqrt(var2 + _EPS)
            h2r = jnp.maximum((h2 - mu2) * sc2 + b2_ref[...], 0.0)
            mx = jnp.max(h2r.reshape(_SQ, _K, 128), axis=1)
            o_ref[0, 0:7, :] = pd_ref[0]
            o_ref[0, 7:135, :] = mx.T


def _mlp(v, qT, pd, w1aT, w1xT, g1r, b1r, w2T, g2r, b2r):
    nt = _TOTAL // _ST
    spb = nt // _B
    return pl.pallas_call(
        _mlp_kern,
        grid=(3, nt),
        in_specs=[
            pl.BlockSpec((_ST, _C), lambda p, s: (s, 0)),
            pl.BlockSpec((1, _SQ, 3), lambda p, s: (s // spb, s % spb, 0)),
            pl.BlockSpec((1, 7, _SQ), lambda p, s: (s // spb, 0, s % spb)),
            pl.BlockSpec((_C, 64), lambda p, s: (0, 0)),
            pl.BlockSpec((3, 64), lambda p, s: (0, 0)),
            pl.BlockSpec((1, 64), lambda p, s: (0, 0)),
            pl.BlockSpec((1, 64), lambda p, s: (0, 0)),
            pl.BlockSpec((64, 128), lambda p, s: (0, 0)),
            pl.BlockSpec((1, 128), lambda p, s: (0, 0)),
            pl.BlockSpec((1, 128), lambda p, s: (0, 0)),
        ],
        out_specs=pl.BlockSpec(
            (1, 135, _SQ),
            lambda p, s: (jnp.where(p < 2, 0, s // spb), 0,
                          jnp.where(p < 2, 0, s % spb))),
        out_shape=jax.ShapeDtypeStruct((_B, 135, _M), jnp.float32),
        scratch_shapes=[
            pltpu.VMEM((1, 64), jnp.float32),
            pltpu.VMEM((1, 64), jnp.float32),
            pltpu.VMEM((1, 128), jnp.float32),
            pltpu.VMEM((1, 128), jnp.float32),
        ],
    )(v, qT, pd, w1aT, w1xT, g1r, b1r, w2T, g2r, b2r)


def kernel(x, W1, g1, b1, W2, g2, b2):
    x3 = x[:, :, :, 0]
    pts = x3[:, 0:3, :]
    qc = x3[:, 0:3, ::_DS]
    qT = jnp.transpose(qc, (0, 2, 1))
    qprep = (qc.reshape(_B, 3, _NW // _B, _QPW)
             .transpose(0, 2, 1, 3).reshape(_NW, 3, _QPW))

    idx = _knn_sc(pts, qprep)

    table = jnp.transpose(x3, (0, 2, 1)).reshape(_B * _N, _C)
    flat_idx = (idx.reshape(_B, _M * _K)
                + (jnp.arange(_B, dtype=jnp.int32) * _N)[:, None]).reshape(-1)
    v = _gather_sc(table, flat_idx)

    w1a = jnp.concatenate(
        [W1[:, 0:6], jnp.zeros((64, 1), jnp.float32), W1[:, 6:15]], axis=1)
    w1aT = w1a.T
    w1xT = W1[:, 0:3].T
    g1r, b1r = g1.reshape(1, 64), b1.reshape(1, 64)
    g2r, b2r = g2.reshape(1, 128), b2.reshape(1, 128)
    w2T = W2.T

    pd = x3[:, 0:_XYZN, ::_DS]
    o = _mlp(v, qT, pd, w1aT, w1xT, g1r, b1r, w2T, g2r, b2r)
    return o[..., None]


# speedup vs baseline: 1.1249x; 1.0050x over previous
import functools

import jax
import jax.numpy as jnp
from jax import lax
from jax.experimental import pallas as pl
from jax.experimental.pallas import tpu as pltpu
from jax.experimental.pallas import tpu_sc as plsc

_B, _C, _N = 2, 16, 8192
_DS = 4
_M = _N // _DS
_K = 32
_XYZN = 7
_EPS = 1e-5
_ST = 16384
_TOTAL = _B * _K * _M
_NW = 32


_QPW = _M * _B // _NW
_QI = 4
_NG = 512
_GV = _NG // 16


def _knn_sc(pts, qprep):
    info = plsc.get_sparse_core_info()
    nc = info.num_cores
    mesh = plsc.VectorSubcoreMesh(core_axis_name="c", subcore_axis_name="s")

    @functools.partial(
        pl.kernel,
        mesh=mesh,
        compiler_params=pltpu.CompilerParams(
            use_tc_tiling_on_sc=False, needs_layout_passes=False),
        out_type=jax.ShapeDtypeStruct((_B * _M * _K,), jnp.int32),
        scratch_types=[
            pltpu.VMEM((3, _N), jnp.float32),
            pltpu.VMEM((3, _QPW), jnp.float32),
            pltpu.VMEM((_QI, _N), jnp.float32),
            pltpu.VMEM((_QI, _NG), jnp.float32),
            pltpu.VMEM((_QI, 32), jnp.float32),
            pltpu.VMEM((_QPW * _K,), jnp.int32),
        ],
    )
    def kk(pts_hbm, q_hbm, out_hbm, ptsv, qv, dbuf, gm, gmm, idxout):
        w = lax.axis_index("s") * nc + lax.axis_index("c")
        b = w // (_NW // _B)
        pltpu.sync_copy(pts_hbm.at[b], ptsv)
        pltpu.sync_copy(q_hbm.at[w], qv)

        iota = lax.broadcasted_iota(jnp.int32, (16,), 0)
        lane0 = iota == 0
        lane1 = iota == 1
        zero16 = jnp.zeros((16,), jnp.int32)
        one16 = jnp.full((16,), 1, jnp.int32)
        two16 = jnp.full((16,), 2, jnp.int32)
        sixteen16 = jnp.full((16,), 16, jnp.int32)
        inf16 = jnp.full((16,), jnp.inf, jnp.float32)
        iota16x = iota * 16
        iota512 = iota * _NG

        def vmin_splat(x):
            nx = -x
            return -plsc.cummax(jnp.flip(plsc.cummax(nx)))

        def per_quad(qp, qis):
            qs = [qis + jnp.full((16,), dq, jnp.int32) for dq in range(_QI)]
            qx = [plsc.load_gather(qv, [zero16, q]) for q in qs]
            qy = [plsc.load_gather(qv, [one16, q]) for q in qs]
            qz = [plsc.load_gather(qv, [two16, q]) for q in qs]

            def dist_chunk(c):
                px = ptsv[0, pl.ds(c * 16, 16)]
                py = ptsv[1, pl.ds(c * 16, 16)]
                pz = ptsv[2, pl.ds(c * 16, 16)]
                ds = []
                for q in range(_QI):
                    dx = qx[q] - px
                    dy = qy[q] - py
                    dz = qz[q] - pz
                    d = dx * dx + dy * dy + dz * dz
                    dbuf[q, pl.ds(c * 16, 16)] = d
                    ds.append(d)
                return tuple(ds)

            def outer(j, vh):
                def inner(k, acc):
                    d = dist_chunk(j + _GV * k)
                    return tuple(jnp.minimum(acc[q], d[q]) for q in range(_QI))

                acc = lax.fori_loop(1, 16, inner, dist_chunk(j), unroll=8)
                for q in range(_QI):
                    gm[q, pl.ds(j * 16, 16)] = acc[q]
                return tuple(jnp.minimum(vh[q], acc[q]) for q in range(_QI))

            va = lax.fori_loop(0, 16, outer, (inf16,) * _QI)
            vb = lax.fori_loop(16, 32, outer, (inf16,) * _QI)
            for q in range(_QI):
                gmm[q, pl.ds(0, 16)] = va[q]
                gmm[q, pl.ds(16, 16)] = vb[q]

            def extract_one(qsel, ks):
                m2a = gmm[qsel, pl.ds(0, 16)]
                m2b = gmm[qsel, pl.ds(16, 16)]
                gmin = vmin_splat(jnp.minimum(m2a, m2b))
                f_a = plsc.all_reduce_ffs(m2a == gmin)
                f_b = plsc.all_reduce_ffs(m2b == gmin)
                isa = f_a < sixteen16
                l2 = jnp.where(isa, f_a, f_b)
                hcell = jnp.where(isa, zero16, sixteen16)
                hbase = hcell * 16
                qsel16 = jnp.full((16,), qsel, jnp.int32)
                gmv = plsc.load_gather(gm, [qsel16, iota16x + hbase + l2])
                jloc = plsc.all_reduce_ffs(gmv == gmin)
                gstar = hbase + jloc * 16 + l2
                midx = iota512 + gstar
                dv = plsc.load_gather(dbuf, [qsel16, midx])
                sd, si = plsc.sort_key_val(dv, midx)
                sgd, _sgi = plsc.sort_key_val(gmv, gmv)
                plsc.store_scatter(idxout, [ks], si, mask=lane0)
                plsc.store_scatter(dbuf, [qsel16, si], inf16, mask=lane0)
                plsc.store_scatter(gm, [qsel16, gstar], sd, mask=lane1)
                plsc.store_scatter(gmm, [qsel16, hcell + l2],
                                   jnp.minimum(sgd, sd), mask=lane1)

            def extract(i, ks):
                for q in range(_QI):
                    extract_one(q, ks + q * _K)
                return ks + 1

            lax.fori_loop(0, _K, extract, qis * _K, unroll=4)
            return qis + _QI

        lax.fori_loop(0, _QPW // _QI, per_quad, zero16)
        pltpu.sync_copy(idxout, out_hbm.at[pl.ds(w * _QPW * _K, _QPW * _K)])

    return kk(pts, qprep)


_CH = 128


def _gather_sc(table, flat_idx):
    per_w = _TOTAL // _NW
    n_ch = per_w // _CH
    info = plsc.get_sparse_core_info()
    nc = info.num_cores
    mesh = plsc.VectorSubcoreMesh(core_axis_name="c", subcore_axis_name="s")

    @functools.partial(
        pl.kernel,
        mesh=mesh,
        compiler_params=pltpu.CompilerParams(
            use_tc_tiling_on_sc=False, needs_layout_passes=False),
        out_type=jax.ShapeDtypeStruct((_TOTAL, _C), jnp.float32),
        scratch_types=[
            pltpu.VMEM((_CH,), jnp.int32),
            pltpu.VMEM((_CH, _C), jnp.float32),
            pltpu.SemaphoreType.DMA,
        ],
    )
    def gk(table_hbm, idx_hbm, out_hbm, idx_v, rows_v, sem):
        wid = lax.axis_index("s") * nc + lax.axis_index("c")

        def body(i, _):
            base = wid * per_w + i * _CH
            pltpu.sync_copy(idx_hbm.at[pl.ds(base, _CH)], idx_v)
            pltpu.async_copy(table_hbm.at[idx_v], rows_v, sem).wait()
            pltpu.sync_copy(rows_v, out_hbm.at[pl.ds(base, _CH)])
            return 0

        lax.fori_loop(0, n_ch, body, 0)

    return gk(table, flat_idx)


_SQ = _ST // _K


def _mlp_kern(v_ref, qT_ref, pd_ref, w1aT_ref, w1xT_ref, g1_ref, b1_ref,
              w2T_ref, g2_ref, b2_ref, o_ref, s1, s2, t1, t2):
    p = pl.program_id(0)
    t = pl.program_id(1)
    h1 = jnp.dot(v_ref[...], w1aT_ref[...], preferred_element_type=jnp.float32)
    pt = jnp.dot(qT_ref[0], w1xT_ref[...], preferred_element_type=jnp.float32)
    h1 = (h1.reshape(_SQ, _K, 64) - pt[:, None, :]).reshape(_ST, 64)

    @pl.when(p == 0)
    def _():
        @pl.when(t == 0)
        def _():
            s1[...] = jnp.zeros_like(s1)
            s2[...] = jnp.zeros_like(s2)

        s1[...] += jnp.sum(h1, axis=0, keepdims=True)
        s2[...] += jnp.sum(h1 * h1, axis=0, keepdims=True)

    @pl.when(p > 0)
    def _():
        mu1 = s1[...] / _TOTAL
        var1 = s2[...] / _TOTAL - mu1 * mu1
        sc1 = g1_ref[...] * lax.rsqrt(var1 + _EPS)
        h1r = jnp.maximum((h1 - mu1) * sc1 + b1_ref[...], 0.0)
        h2 = jnp.dot(h1r, w2T_ref[...], preferred_element_type=jnp.float32)

        @pl.when(p == 1)
        def _():
            @pl.when(t == 0)
            def _():
                t1[...] = jnp.zeros_like(t1)
                t2[...] = jnp.zeros_like(t2)

            t1[...] += jnp.sum(h2, axis=0, keepdims=True)
            t2[...] += jnp.sum(h2 * h2, axis=0, keepdims=True)

        @pl.when(p == 2)
        def _():
            mu2 = t1[...] / _TOTAL
            var2 = t2[...] / _TOTAL - mu2 * mu2
            sc2 = g2_ref[...] * lax.rsqrt(var2 + _EPS)
            h2r = jnp.maximum((h2 - mu2) * sc2 + b2_ref[...], 0.0)
            mx = jnp.max(h2r.reshape(_SQ, _K, 128), axis=1)
            o_ref[0, 0:7, :] = pd_ref[0]
            o_ref[0, 7:135, :] = mx.T


def _mlp(v, qT, pd, w1aT, w1xT, g1r, b1r, w2T, g2r, b2r):
    nt = _TOTAL // _ST
    spb = nt // _B
    return pl.pallas_call(
        _mlp_kern,
        grid=(3, nt),
        in_specs=[
            pl.BlockSpec((_ST, _C), lambda p, s: (s, 0)),
            pl.BlockSpec((1, _SQ, 3), lambda p, s: (s // spb, s % spb, 0)),
            pl.BlockSpec((1, 7, _SQ), lambda p, s: (s // spb, 0, s % spb)),
            pl.BlockSpec((_C, 64), lambda p, s: (0, 0)),
            pl.BlockSpec((3, 64), lambda p, s: (0, 0)),
            pl.BlockSpec((1, 64), lambda p, s: (0, 0)),
            pl.BlockSpec((1, 64), lambda p, s: (0, 0)),
            pl.BlockSpec((64, 128), lambda p, s: (0, 0)),
            pl.BlockSpec((1, 128), lambda p, s: (0, 0)),
            pl.BlockSpec((1, 128), lambda p, s: (0, 0)),
        ],
        out_specs=pl.BlockSpec(
            (1, 135, _SQ),
            lambda p, s: (jnp.where(p < 2, 0, s // spb), 0,
                          jnp.where(p < 2, 0, s % spb))),
        out_shape=jax.ShapeDtypeStruct((_B, 135, _M), jnp.float32),
        scratch_shapes=[
            pltpu.VMEM((1, 64), jnp.float32),
            pltpu.VMEM((1, 64), jnp.float32),
            pltpu.VMEM((1, 128), jnp.float32),
            pltpu.VMEM((1, 128), jnp.float32),
        ],
    )(v, qT, pd, w1aT, w1xT, g1r, b1r, w2T, g2r, b2r)


def kernel(x, W1, g1, b1, W2, g2, b2):
    x3 = x[:, :, :, 0]
    pts = x3[:, 0:3, :]
    qc = x3[:, 0:3, ::_DS]
    qT = jnp.transpose(qc, (0, 2, 1))
    qprep = (qc.reshape(_B, 3, _NW // _B, _QPW)
             .transpose(0, 2, 1, 3).reshape(_NW, 3, _QPW))

    idx = _knn_sc(pts, qprep)

    table = jnp.transpose(x3, (0, 2, 1)).reshape(_B * _N, _C)
    flat_idx = (idx.reshape(_B, _M * _K)
                + (jnp.arange(_B, dtype=jnp.int32) * _N)[:, None]).reshape(-1)
    v = _gather_sc(table, flat_idx)

    w1a = jnp.concatenate(
        [W1[:, 0:6], jnp.zeros((64, 1), jnp.float32), W1[:, 6:15]], axis=1)
    w1aT = w1a.T
    w1xT = W1[:, 0:3].T
    g1r, b1r = g1.reshape(1, 64), b1.reshape(1, 64)
    g2r, b2r = g2.reshape(1, 128), b2.reshape(1, 128)
    w2T = W2.T

    pd = x3[:, 0:_XYZN, ::_DS]
    o = _mlp(v, qT, pd, w1aT, w1xT, g1r, b1r, w2T, g2r, b2r)
    return o[..., None]
